# baseline (speedup 1.0000x reference)
import jax
import jax.numpy as jnp
from jax.experimental import pallas as pl
from jax.experimental.pallas import tpu as pltpu

_F32 = jnp.float32
_BF16 = jnp.bfloat16
_T = 1024
_K = 51
_HI = 1 << 30


def _sigmoid(z):
    return 1.0 / (1.0 + jnp.exp(-z))


def _dot_f32(a, b):
    return jax.lax.dot_general(a, b, (((1,), (0,)), ((), ())),
                               precision=jax.lax.Precision.HIGHEST,
                               preferred_element_type=_F32)


def _dot_bf16(a, b):
    return jax.lax.dot_general(a.astype(_BF16), b.astype(_BF16),
                               (((1,), (0,)), ((), ())),
                               preferred_element_type=_F32)


def _conv_body(xb_ref, xrf_ref, wb_ref, wrf_ref, bb_ref, brg_ref, bfl_ref,
               emb_ref, embr_ref, embf_ref, accb_ref, accrf_ref):
    s = pl.program_id(1)
    db = _dot_bf16(xb_ref[0, 0], wb_ref[...])
    drf = _dot_bf16(xrf_ref[0, 0], wrf_ref[...])

    @pl.when(s == 0)
    def _():
        accb_ref[...] = db
        accrf_ref[...] = drf

    @pl.when((s > 0) & (s < 11))
    def _():
        accb_ref[...] += db

    @pl.when((s > 0) & (s < 5))
    def _():
        accrf_ref[...] += drf

    @pl.when(s == 5)
    def _():
        embr_ref[0] = jnp.maximum(accrf_ref[...] + drf + brg_ref[...], 0.0)

    @pl.when(s == 6)
    def _():
        accrf_ref[...] = drf

    @pl.when((s > 6) & (s < 11))
    def _():
        accrf_ref[...] += drf

    @pl.when(s == 11)
    def _():
        emb_ref[0] = jnp.maximum(accb_ref[...] + db + bb_ref[...], 0.0)
        embf_ref[0] = jnp.maximum(accrf_ref[...] + drf + bfl_ref[...], 0.0)


def _kth_largest_val(bits, k):
    def step(_, lohi):
        lo, hi = lohi
        mid = (lo + hi) // 2
        cnt = jnp.sum((bits >= mid).astype(jnp.int32))
        ge = cnt >= k
        return (jnp.where(ge, mid, lo), jnp.where(ge, hi, mid))
    lo, _ = jax.lax.fori_loop(0, 31, step, (jnp.int32(0), jnp.int32(_HI)))
    v = jax.lax.bitcast_convert_type(jnp.full((8, 128), lo, jnp.int32), _F32)
    return jnp.max(v)


def _median(a8):
    bits = jax.lax.bitcast_convert_type(a8, jnp.int32)
    return (_kth_largest_val(bits, 512) + _kth_largest_val(bits, 513)) * 0.5


def _sel_body(emb_ref, embr_ref, embf_ref, wc_ref, wcr_ref, wcf_ref,
              bc_ref, br_ref, bf_ref,
              cas_ref, af_ref, ar_ref,
              ca_ref, cb_ref, ia_ref, ib_ref,
              easy_ref, cidx_ref,
              car_ref, cbr_ref, iar_ref, ibr_ref,
              caf_ref, cbf_ref, iaf_ref, ibf_ref,
              a1_ref, a2_ref, bin1_ref, bin2_ref,
              oca_ref, ocb_ref, oia_ref, oib_ref):
    cas = _dot_bf16(emb_ref[0], wc_ref[...]) + bc_ref[...]
    cas_ref[0] = cas

    a1c = _sigmoid(jnp.sum(cas, axis=1, keepdims=True))
    arc = _sigmoid(_dot_bf16(embr_ref[0], wcr_ref[...]) + br_ref[0, 0])
    afc = _sigmoid(_dot_bf16(embf_ref[0], wcf_ref[...]) + bf_ref[0, 0])
    af_ref[0] = afc
    ar_ref[0] = arc

    a1 = a1c.reshape(8, 128)
    a2 = (afc.reshape(8, 128) + arc.reshape(8, 128)) * 0.5
    a1_ref[0] = a1
    a2_ref[0] = a2

    bin1 = jnp.where(a1 >= _median(a1), 1.0, 0.0).astype(_F32)
    bin2 = jnp.where(a2 >= _median(a2), 1.0, 0.0).astype(_F32)
    bin1_ref[0] = bin1
    bin2_ref[0] = bin2

    xm = bin1 + bin2
    m_act = (xm == 2.0).astype(_F32)
    m_bg = (xm == 0.0).astype(_F32)
    m_in = (xm == 1.0).astype(_F32)
    cidx_ref[0] = m_act

    combined = (a1 + a2) * 0.5
    rev1 = jnp.max(a1) - a1
    s0_orig = combined * m_act
    scores = [s0_orig, a1 * m_act, rev1 * m_bg, a1 * m_in, rev1 * m_in]

    t2d = (jax.lax.broadcasted_iota(jnp.int32, (8, 128), 0) * 128
           + jax.lax.broadcasted_iota(jnp.int32, (8, 128), 1))
    row_iota = jax.lax.broadcasted_iota(jnp.int32, (1, _T), 1)
    o_refs = [None, oca_ref, ocb_ref, oia_ref, oib_ref]

    def pick_step(j, carry):
        s_list = list(carry[:5])
        mem = carry[5]
        for g in range(5):
            s = s_list[g]
            m = jnp.max(s)
            pg = jnp.min(jnp.where(s == m, t2d, jnp.int32(4096)))
            s_list[g] = jnp.where(t2d == pg, -1.0, s)
            if g == 0:
                mem = jnp.where(t2d == pg, 1.0, mem)
            else:
                o_refs[g][pl.ds(j, 1), :] = (row_iota == pg).astype(_F32)
        return tuple(s_list) + (mem,)

    init = tuple(scores) + (jnp.zeros((8, 128), _F32),)
    out = jax.lax.fori_loop(0, _K, pick_step, init)
    easy_ref[0] = s0_orig * out[5]

    for o_ref, outs in ((oca_ref, (ca_ref, car_ref, caf_ref)),
                        (ocb_ref, (cb_ref, cbr_ref, cbf_ref)),
                        (oia_ref, (ia_ref, iar_ref, iaf_ref)),
                        (oib_ref, (ib_ref, ibr_ref, ibf_ref))):
        onehot = o_ref[...]
        for dst, table in zip(outs, (emb_ref, embr_ref, embf_ref)):
            dst[0] = _dot_f32(onehot, table[0])[:_K, :]


def _im2col_w(w):
    return jnp.concatenate([w[:, :, h].T for h in range(3)], axis=0)


@jax.jit
def kernel(x, w_base, b_base, w_cls, b_cls, w_rgb, b_rgb, w_cls_rgb,
           b_cls_rgb, w_flow, b_flow, w_cls_flow, b_cls_flow):
    B = x.shape[0]
    xbf = x.astype(_BF16)
    zrow = jnp.zeros((B, 1, 2048), _BF16)
    xs3 = jnp.stack([
        jnp.concatenate([zrow, xbf[:, :-1, :]], axis=1),
        xbf,
        jnp.concatenate([xbf[:, 1:, :], zrow], axis=1),
    ])
    wb = _im2col_w(w_base).astype(_BF16)
    wrf = jnp.concatenate([_im2col_w(w_rgb),
                           _im2col_w(w_flow)], axis=0).astype(_BF16)
    wc = w_cls[:, :, 0].T
    wcr = w_cls_rgb[0, :, 0][:, None]
    wcf = w_cls_flow[0, :, 0][:, None]

    def rf_map(b, s):
        rgb = s < 6
        h = jnp.where(rgb, s // 2, (s - 6) // 2)
        c = jnp.where(rgb, s % 2, 2 + (s - 6) % 2)
        return (h, b, 0, c)

    emb_sds = jax.ShapeDtypeStruct((B, _T, 512), _F32)
    emb, embr, embf = pl.pallas_call(
        _conv_body,
        grid=(B, 12),
        in_specs=[
            pl.BlockSpec((1, 1, _T, 512), lambda b, s: (s // 4, b, 0, s % 4)),
            pl.BlockSpec((1, 1, _T, 512), rf_map),
            pl.BlockSpec((512, 512), lambda b, s: (s, 0)),
            pl.BlockSpec((512, 512), lambda b, s: (s, 0)),
            pl.BlockSpec((1, 512), lambda b, s: (0, 0)),
            pl.BlockSpec((1, 512), lambda b, s: (0, 0)),
            pl.BlockSpec((1, 512), lambda b, s: (0, 0)),
        ],
        out_specs=[pl.BlockSpec((1, _T, 512), lambda b, s: (b, 0, 0))] * 3,
        out_shape=[emb_sds] * 3,
        scratch_shapes=[pltpu.VMEM((_T, 512), _F32)] * 2,
        compiler_params=pltpu.CompilerParams(
            dimension_semantics=("arbitrary", "arbitrary"),
            vmem_limit_bytes=63 * 1024 * 1024,
        ),
    )(xs3, xs3, wb, wrf,
      b_base[None, :], b_rgb[None, :], b_flow[None, :])

    full = lambda *shape: pl.BlockSpec(shape, lambda b: (0,) * len(shape))
    perb = lambda *shape: pl.BlockSpec((1,) + shape, lambda b: (b,) + (0,) * len(shape))
    emb_out = [jax.ShapeDtypeStruct((B, _K, 512), _F32)] * 4
    o8 = jax.ShapeDtypeStruct((B, 8, 128), _F32)

    outs = pl.pallas_call(
        _sel_body,
        grid=(B,),
        in_specs=[
            perb(_T, 512), perb(_T, 512), perb(_T, 512),
            full(512, 100), full(512, 1), full(512, 1),
            full(1, 100), full(1, 1), full(1, 1),
        ],
        out_specs=[
            perb(_T, 100), perb(_T, 1), perb(_T, 1),
            perb(_K, 512), perb(_K, 512), perb(_K, 512), perb(_K, 512),
            perb(8, 128), perb(8, 128),
            perb(_K, 512), perb(_K, 512), perb(_K, 512), perb(_K, 512),
            perb(_K, 512), perb(_K, 512), perb(_K, 512), perb(_K, 512),
            perb(8, 128), perb(8, 128), perb(8, 128), perb(8, 128),
        ],
        out_shape=[
            jax.ShapeDtypeStruct((B, _T, 100), _F32),
            jax.ShapeDtypeStruct((B, _T, 1), _F32),
            jax.ShapeDtypeStruct((B, _T, 1), _F32),
            *emb_out, o8, o8, *emb_out, *emb_out,
            o8, o8, o8, o8,
        ],
        scratch_shapes=[pltpu.VMEM((64, _T), _F32)] * 4,
        compiler_params=pltpu.CompilerParams(
            dimension_semantics=("arbitrary",),
            vmem_limit_bytes=63 * 1024 * 1024,
        ),
    )(emb, embr, embf, wc, wcr, wcf,
      b_cls[None, :], b_cls_rgb[None, :], b_cls_flow[None, :])

    (cas, afc, arc, ca, cb, ia, ib, easy, cidx,
     car, cbr, iar, ibr, caf, cbf, iaf, ibf,
     a1, a2, bin1, bin2) = outs

    flat = lambda a: a.reshape(B, _T)
    return (cas, afc.reshape(B, 1, _T), arc.reshape(B, 1, _T),
            ca, cb, ia, ib, flat(easy), flat(cidx),
            car, cbr, iar, ibr, caf, cbf, iaf, ibf,
            flat(a1), flat(a2), flat(bin1), flat(bin2))


# speedup vs baseline: 1.2708x; 1.2708x over previous
import jax
import jax.numpy as jnp
from jax.experimental import pallas as pl
from jax.experimental.pallas import tpu as pltpu

_F32 = jnp.float32
_BF16 = jnp.bfloat16
_T = 1024
_K = 51
_HI = 1 << 30


def _sigmoid(z):
    return 1.0 / (1.0 + jnp.exp(-z))


def _dot_f32(a, b):
    return jax.lax.dot_general(a, b, (((1,), (0,)), ((), ())),
                               precision=jax.lax.Precision.HIGHEST,
                               preferred_element_type=_F32)


def _dot_bf16(a, b):
    return jax.lax.dot_general(a.astype(_BF16), b.astype(_BF16),
                               (((1,), (0,)), ((), ())),
                               preferred_element_type=_F32)


def _shift_chunk(xk, h):
    z = jnp.zeros((1, 512), _F32)
    if h == 0:
        return jnp.concatenate([z, xk[:-1, :]], axis=0)
    if h == 2:
        return jnp.concatenate([xk[1:, :], z], axis=0)
    return xk


def _conv_body(x_ref, wb_ref, wrf_ref, bb_ref, brg_ref, bfl_ref,
               emb_ref, embr_ref, embf_ref):
    x = x_ref[0]
    chunks = [x[:, c * 512:(c + 1) * 512] for c in range(4)]

    accb = None
    for h in range(3):
        for c in range(4):
            d = _dot_bf16(_shift_chunk(chunks[c], h), wb_ref[(h * 4 + c) * 512:(h * 4 + c + 1) * 512, :])
            accb = d if accb is None else accb + d
    emb_ref[0] = jnp.maximum(accb + bb_ref[...], 0.0)

    accr = None
    for h in range(3):
        for c in range(2):
            d = _dot_bf16(_shift_chunk(chunks[c], h), wrf_ref[(h * 2 + c) * 512:(h * 2 + c + 1) * 512, :])
            accr = d if accr is None else accr + d
    embr_ref[0] = jnp.maximum(accr + brg_ref[...], 0.0)

    accf = None
    for h in range(3):
        for c in range(2):
            d = _dot_bf16(_shift_chunk(chunks[2 + c], h), wrf_ref[3072 + (h * 2 + c) * 512:3072 + (h * 2 + c + 1) * 512, :])
            accf = d if accf is None else accf + d
    embf_ref[0] = jnp.maximum(accf + bfl_ref[...], 0.0)


def _kth_largest_val(bits, k):
    def step(_, lohi):
        lo, hi = lohi
        mid = (lo + hi) // 2
        cnt = jnp.sum((bits >= mid).astype(jnp.int32))
        ge = cnt >= k
        return (jnp.where(ge, mid, lo), jnp.where(ge, hi, mid))
    lo, _ = jax.lax.fori_loop(0, 31, step, (jnp.int32(0), jnp.int32(_HI)))
    v = jax.lax.bitcast_convert_type(jnp.full((8, 128), lo, jnp.int32), _F32)
    return jnp.max(v)


def _median(a8):
    bits = jax.lax.bitcast_convert_type(a8, jnp.int32)
    return (_kth_largest_val(bits, 512) + _kth_largest_val(bits, 513)) * 0.5


def _sel_body(emb_ref, embr_ref, embf_ref, wc_ref, wcr_ref, wcf_ref,
              bc_ref, br_ref, bf_ref,
              cas_ref, af_ref, ar_ref,
              ca_ref, cb_ref, ia_ref, ib_ref,
              easy_ref, cidx_ref,
              car_ref, cbr_ref, iar_ref, ibr_ref,
              caf_ref, cbf_ref, iaf_ref, ibf_ref,
              a1_ref, a2_ref, bin1_ref, bin2_ref,
              oca_ref, ocb_ref, oia_ref, oib_ref):
    cas = _dot_bf16(emb_ref[0], wc_ref[...]) + bc_ref[...]
    cas_ref[0] = cas

    a1c = _sigmoid(jnp.sum(cas, axis=1, keepdims=True))
    arc = _sigmoid(_dot_bf16(embr_ref[0], wcr_ref[...]) + br_ref[0, 0])
    afc = _sigmoid(_dot_bf16(embf_ref[0], wcf_ref[...]) + bf_ref[0, 0])
    af_ref[0] = afc
    ar_ref[0] = arc

    a1 = a1c.reshape(8, 128)
    a2 = (afc.reshape(8, 128) + arc.reshape(8, 128)) * 0.5
    a1_ref[0] = a1
    a2_ref[0] = a2

    bin1 = jnp.where(a1 >= _median(a1), 1.0, 0.0).astype(_F32)
    bin2 = jnp.where(a2 >= _median(a2), 1.0, 0.0).astype(_F32)
    bin1_ref[0] = bin1
    bin2_ref[0] = bin2

    xm = bin1 + bin2
    m_act = (xm == 2.0).astype(_F32)
    m_bg = (xm == 0.0).astype(_F32)
    m_in = (xm == 1.0).astype(_F32)
    cidx_ref[0] = m_act

    combined = (a1 + a2) * 0.5
    rev1 = jnp.max(a1) - a1
    s0_orig = combined * m_act
    scores = [s0_orig, a1 * m_act, rev1 * m_bg, a1 * m_in, rev1 * m_in]

    t2d = (jax.lax.broadcasted_iota(jnp.int32, (8, 128), 0) * 128
           + jax.lax.broadcasted_iota(jnp.int32, (8, 128), 1))
    row_iota = jax.lax.broadcasted_iota(jnp.int32, (1, _T), 1)
    o_refs = [None, oca_ref, ocb_ref, oia_ref, oib_ref]

    def pick_step(j, carry):
        s_list = list(carry[:5])
        mem = carry[5]
        for g in range(5):
            s = s_list[g]
            m = jnp.max(s)
            pg = jnp.min(jnp.where(s == m, t2d, jnp.int32(4096)))
            s_list[g] = jnp.where(t2d == pg, -1.0, s)
            if g == 0:
                mem = jnp.where(t2d == pg, 1.0, mem)
            else:
                o_refs[g][pl.ds(j, 1), :] = (row_iota == pg).astype(_F32)
        return tuple(s_list) + (mem,)

    init = tuple(scores) + (jnp.zeros((8, 128), _F32),)
    out = jax.lax.fori_loop(0, _K, pick_step, init)
    easy_ref[0] = s0_orig * out[5]

    for o_ref, outs in ((oca_ref, (ca_ref, car_ref, caf_ref)),
                        (ocb_ref, (cb_ref, cbr_ref, cbf_ref)),
                        (oia_ref, (ia_ref, iar_ref, iaf_ref)),
                        (oib_ref, (ib_ref, ibr_ref, ibf_ref))):
        onehot = o_ref[...]
        for dst, table in zip(outs, (emb_ref, embr_ref, embf_ref)):
            dst[0] = _dot_f32(onehot, table[0])[:_K, :]


def _im2col_w(w):
    return jnp.concatenate([w[:, :, h].T for h in range(3)], axis=0)


@jax.jit
def kernel(x, w_base, b_base, w_cls, b_cls, w_rgb, b_rgb, w_cls_rgb,
           b_cls_rgb, w_flow, b_flow, w_cls_flow, b_cls_flow):
    B = x.shape[0]
    wb = _im2col_w(w_base).astype(_BF16)
    wrf = jnp.concatenate([_im2col_w(w_rgb),
                           _im2col_w(w_flow)], axis=0).astype(_BF16)
    wc = w_cls[:, :, 0].T
    wcr = w_cls_rgb[0, :, 0][:, None]
    wcf = w_cls_flow[0, :, 0][:, None]

    emb_sds = jax.ShapeDtypeStruct((B, _T, 512), _F32)
    emb, embr, embf = pl.pallas_call(
        _conv_body,
        grid=(B,),
        in_specs=[
            pl.BlockSpec((1, _T, 2048), lambda b: (b, 0, 0)),
            pl.BlockSpec((6144, 512), lambda b: (0, 0)),
            pl.BlockSpec((6144, 512), lambda b: (0, 0)),
            pl.BlockSpec((1, 512), lambda b: (0, 0)),
            pl.BlockSpec((1, 512), lambda b: (0, 0)),
            pl.BlockSpec((1, 512), lambda b: (0, 0)),
        ],
        out_specs=[pl.BlockSpec((1, _T, 512), lambda b: (b, 0, 0))] * 3,
        out_shape=[emb_sds] * 3,
        compiler_params=pltpu.CompilerParams(
            dimension_semantics=("arbitrary",),
            vmem_limit_bytes=63 * 1024 * 1024,
        ),
    )(x, wb, wrf,
      b_base[None, :], b_rgb[None, :], b_flow[None, :])

    full = lambda *shape: pl.BlockSpec(shape, lambda b: (0,) * len(shape))
    perb = lambda *shape: pl.BlockSpec((1,) + shape, lambda b: (b,) + (0,) * len(shape))
    emb_out = [jax.ShapeDtypeStruct((B, _K, 512), _F32)] * 4
    o8 = jax.ShapeDtypeStruct((B, 8, 128), _F32)

    outs = pl.pallas_call(
        _sel_body,
        grid=(B,),
        in_specs=[
            perb(_T, 512), perb(_T, 512), perb(_T, 512),
            full(512, 100), full(512, 1), full(512, 1),
            full(1, 100), full(1, 1), full(1, 1),
        ],
        out_specs=[
            perb(_T, 100), perb(_T, 1), perb(_T, 1),
            perb(_K, 512), perb(_K, 512), perb(_K, 512), perb(_K, 512),
            perb(8, 128), perb(8, 128),
            perb(_K, 512), perb(_K, 512), perb(_K, 512), perb(_K, 512),
            perb(_K, 512), perb(_K, 512), perb(_K, 512), perb(_K, 512),
            perb(8, 128), perb(8, 128), perb(8, 128), perb(8, 128),
        ],
        out_shape=[
            jax.ShapeDtypeStruct((B, _T, 100), _F32),
            jax.ShapeDtypeStruct((B, _T, 1), _F32),
            jax.ShapeDtypeStruct((B, _T, 1), _F32),
            *emb_out, o8, o8, *emb_out, *emb_out,
            o8, o8, o8, o8,
        ],
        scratch_shapes=[pltpu.VMEM((64, _T), _F32)] * 4,
        compiler_params=pltpu.CompilerParams(
            dimension_semantics=("arbitrary",),
            vmem_limit_bytes=63 * 1024 * 1024,
        ),
    )(emb, embr, embf, wc, wcr, wcf,
      b_cls[None, :], b_cls_rgb[None, :], b_cls_flow[None, :])

    (cas, afc, arc, ca, cb, ia, ib, easy, cidx,
     car, cbr, iar, ibr, caf, cbf, iaf, ibf,
     a1, a2, bin1, bin2) = outs

    flat = lambda a: a.reshape(B, _T)
    return (cas, afc.reshape(B, 1, _T), arc.reshape(B, 1, _T),
            ca, cb, ia, ib, flat(easy), flat(cidx),
            car, cbr, iar, ibr, caf, cbf, iaf, ibf,
            flat(a1), flat(a2), flat(bin1), flat(bin2))
